# baseline (speedup 1.0000x reference)
import functools

import jax
import jax.numpy as jnp
from jax import lax
from jax.experimental import pallas as pl
from jax.experimental.pallas import tpu as pltpu
from jax.experimental.pallas import tpu_sc as plsc

_B = 16384
_D = 32
_LANES = 16
_PLANE = 8
_NPLANE = _D // _PLANE

_NC = 2
_NS = 16
_NW = _NC * _NS
_BPW = _B // _NW
_NCHUNK = 2
_CPW = _BPW // _NCHUNK

_GATHER_DN = lax.GatherDimensionNumbers(
    offset_dims=(), collapsed_slice_dims=(0,), start_index_map=(0,))


def _lane_perm(v, idx):
    return lax.gather(v, idx[:, None], _GATHER_DN, (1,),
                      mode=lax.GatherScatterMode.PROMISE_IN_BOUNDS)


def _sc_body(n_users, n_items, uidx_hbm, iidx_hbm, utab_hbm, itab_hbm,
             out_hbm, uidx_v, iidx_v, uq_v, iq_v, up2_v, ip2_v, us8_v, is8_v,
             urows_v, irows_v, out_v, sem):
    wid = lax.axis_index("s") * _NC + lax.axis_index("c")
    base = wid * _BPW
    urows_per_plane = n_users // 4
    irows_per_plane = n_items // 4

    pltpu.sync_copy(uidx_hbm.at[pl.ds(base, _BPW)], uidx_v)
    pltpu.sync_copy(iidx_hbm.at[pl.ds(base, _BPW)], iidx_v)

    def split(g, carry):
        gbase = g * _LANES
        xu = uidx_v[pl.ds(gbase, _LANES)]
        xi = iidx_v[pl.ds(gbase, _LANES)]
        qu = lax.shift_right_logical(xu, 2)
        qi = lax.shift_right_logical(xi, 2)
        for k in range(_NPLANE):
            uq_v[k, pl.ds(gbase, _LANES)] = qu + k * urows_per_plane
            iq_v[k, pl.ds(gbase, _LANES)] = qi + k * irows_per_plane
        up2_v[pl.ds(gbase, _LANES)] = (xu & 2) * 8
        ip2_v[pl.ds(gbase, _LANES)] = (xi & 2) * 8
        us8_v[pl.ds(gbase, _LANES)] = (xu & 1) * 8
        is8_v[pl.ds(gbase, _LANES)] = (xi & 1) * 8
        return carry

    lax.fori_loop(0, _BPW // _LANES, split, 0)

    lane_iota = lax.iota(jnp.int32, _LANES)
    lane_lo8 = lane_iota & 7

    for c in range(_NCHUNK):
        copies = []
        for k in range(_NPLANE):
            copies.append(pltpu.async_copy(
                utab_hbm.at[uq_v.at[k, pl.ds(c * _CPW, _CPW)]],
                urows_v.at[k], sem))
            copies.append(pltpu.async_copy(
                itab_hbm.at[iq_v.at[k, pl.ds(c * _CPW, _CPW)]],
                irows_v.at[k], sem))
        for cp in copies:
            cp.wait()

        def group(g, carry):
            gbase = g * _LANES
            p2u = up2_v[pl.ds(c * _CPW + gbase, _LANES)]
            p2i = ip2_v[pl.ds(c * _CPW + gbase, _LANES)]
            s8u = us8_v[pl.ds(c * _CPW + gbase, _LANES)]
            s8i = is8_v[pl.ds(c * _CPW + gbase, _LANES)]
            acc = jnp.zeros((_LANES,), jnp.float32)
            for j in range(_LANES):
                r = gbase + j
                permu = lane_lo8 + s8u[j]
                permi = lane_lo8 + s8i[j]
                prod = jnp.zeros((_LANES,), jnp.float32)
                for k in range(_NPLANE):
                    uv = urows_v[k, r, pl.ds(p2u[j], _LANES)]
                    iv = irows_v[k, r, pl.ds(p2i[j], _LANES)]
                    ua = _lane_perm(uv, permu)
                    ia = _lane_perm(iv, permi)
                    prod = prod + ua * ia
                s = jnp.cumsum(prod)[_PLANE - 1]
                acc = jnp.where(lane_iota == j, s, acc)
            out_v[pl.ds(c * _CPW + gbase, _LANES)] = (
                1.0 / (1.0 + jnp.exp(-acc)))
            return carry

        lax.fori_loop(0, _CPW // _LANES, group, 0)

    pltpu.sync_copy(out_v, out_hbm.at[pl.ds(base, _BPW)])


@jax.jit
def _mf_forward(user_indices, item_indices, user_planes, item_planes):
    n_users = user_planes.shape[0]
    n_items = item_planes.shape[0]
    mesh = plsc.VectorSubcoreMesh(core_axis_name="c", subcore_axis_name="s")
    run = functools.partial(
        pl.kernel,
        mesh=mesh,
        compiler_params=pltpu.CompilerParams(
            needs_layout_passes=False, use_tc_tiling_on_sc=False
        ),
        out_type=jax.ShapeDtypeStruct((_B,), jnp.float32),
        scratch_types=[
            pltpu.VMEM((_BPW,), jnp.int32),
            pltpu.VMEM((_BPW,), jnp.int32),
            pltpu.VMEM((_NPLANE, _BPW), jnp.int32),
            pltpu.VMEM((_NPLANE, _BPW), jnp.int32),
            pltpu.VMEM((_BPW,), jnp.int32),
            pltpu.VMEM((_BPW,), jnp.int32),
            pltpu.VMEM((_BPW,), jnp.int32),
            pltpu.VMEM((_BPW,), jnp.int32),
            pltpu.VMEM((_NPLANE, _CPW, _D), jnp.float32),
            pltpu.VMEM((_NPLANE, _CPW, _D), jnp.float32),
            pltpu.VMEM((_BPW,), jnp.float32),
            pltpu.SemaphoreType.DMA,
        ],
    )(functools.partial(_sc_body, n_users, n_items))
    return run(user_indices, item_indices, user_planes, item_planes)


def _restack(table):
    n, d = table.shape
    planes = jnp.concatenate(
        [table[:, k * _PLANE:(k + 1) * _PLANE] for k in range(_NPLANE)],
        axis=0)
    return planes.reshape(n, d)


def kernel(user_indices, item_indices, user_table, item_table):
    one = jnp.where(user_indices[0] < 0, jnp.float32(2.0), jnp.float32(1.0))
    return _mf_forward(
        user_indices.astype(jnp.int32),
        item_indices.astype(jnp.int32),
        _restack(user_table) * one,
        _restack(item_table) * one,
    )


# speedup vs baseline: 6.2294x; 6.2294x over previous
import functools

import jax
import jax.numpy as jnp
from jax import lax
from jax.experimental import pallas as pl
from jax.experimental.pallas import tpu as pltpu
from jax.experimental.pallas import tpu_sc as plsc

_B = 16384
_D = 32
_LANES = 16

_NC = 2
_NS = 16
_NW = _NC * _NS
_BPW = _B // _NW


def _sc_body(uidx_hbm, iidx_hbm, utab_hbm, itab_hbm, out_hbm,
             uidx_v, iidx_v, urows_v, irows_v, out_v, sem):
    wid = lax.axis_index("s") * _NC + lax.axis_index("c")
    base = wid * _BPW

    pltpu.sync_copy(uidx_hbm.at[pl.ds(base, _BPW)], uidx_v)
    pltpu.sync_copy(iidx_hbm.at[pl.ds(base, _BPW)], iidx_v)
    cu = pltpu.async_copy(utab_hbm.at[uidx_v], urows_v, sem)
    ci = pltpu.async_copy(itab_hbm.at[iidx_v], irows_v, sem)
    cu.wait()
    ci.wait()

    lane_iota = lax.iota(jnp.int32, _LANES)

    def group(g, carry):
        gbase = g * _LANES
        acc = jnp.zeros((_LANES,), jnp.float32)
        for j in range(_LANES):
            r = gbase + j
            u0 = urows_v[r, pl.ds(0, _LANES)]
            u1 = urows_v[r, pl.ds(_LANES, _LANES)]
            i0 = irows_v[r, pl.ds(0, _LANES)]
            i1 = irows_v[r, pl.ds(_LANES, _LANES)]
            s = jnp.sum(u0 * i0 + u1 * i1)
            acc = jnp.where(lane_iota == j, s, acc)
        out_v[pl.ds(gbase, _LANES)] = 1.0 / (1.0 + jnp.exp(-acc))
        return carry

    lax.fori_loop(0, _BPW // _LANES, group, 0)

    pltpu.sync_copy(out_v, out_hbm.at[pl.ds(base, _BPW)])


@jax.jit
def _mf_forward(user_indices, item_indices, user_table, item_table):
    mesh = plsc.VectorSubcoreMesh(core_axis_name="c", subcore_axis_name="s")
    run = functools.partial(
        pl.kernel,
        mesh=mesh,
        compiler_params=pltpu.CompilerParams(
            needs_layout_passes=False, use_tc_tiling_on_sc=False
        ),
        out_type=jax.ShapeDtypeStruct((_B,), jnp.float32),
        scratch_types=[
            pltpu.VMEM((_BPW,), jnp.int32),
            pltpu.VMEM((_BPW,), jnp.int32),
            pltpu.VMEM((_BPW, _D), jnp.float32),
            pltpu.VMEM((_BPW, _D), jnp.float32),
            pltpu.VMEM((_BPW,), jnp.float32),
            pltpu.SemaphoreType.DMA,
        ],
    )(_sc_body)
    return run(user_indices, item_indices, user_table, item_table)


def kernel(user_indices, item_indices, user_table, item_table):
    return _mf_forward(
        user_indices.astype(jnp.int32),
        item_indices.astype(jnp.int32),
        user_table,
        item_table,
    )
